# baseline (speedup 1.0000x reference)
import functools

import jax
import jax.numpy as jnp
from jax import lax
from jax.experimental import pallas as pl
from jax.experimental.pallas import tpu as pltpu
from jax.experimental.pallas import tpu_sc as plsc

_D_X = 1024
_DIM = 128


def kernel(x, pe):
    b, s, d_x = x.shape
    size, dim = pe.shape
    rows = b * s
    x2 = x.reshape(rows, d_x)
    info = plsc.get_sparse_core_info()
    nw = info.num_cores * info.num_subcores
    rows_per_w = rows // nw
    mesh = plsc.VectorSubcoreMesh(core_axis_name="c", subcore_axis_name="s")

    @functools.partial(
        pl.kernel,
        mesh=mesh,
        out_type=jax.ShapeDtypeStruct((rows, d_x + dim), x.dtype),
    )
    def k(x_hbm, pe_hbm, out_hbm):
        wid = lax.axis_index("s") * info.num_cores + lax.axis_index("c")
        base = wid * rows_per_w
        s_base = lax.rem(base, size)
        pltpu.sync_copy(
            x_hbm.at[pl.ds(base, rows_per_w), :],
            out_hbm.at[pl.ds(base, rows_per_w), pl.ds(0, d_x)],
        )
        pltpu.sync_copy(
            pe_hbm.at[pl.ds(s_base, rows_per_w), :],
            out_hbm.at[pl.ds(base, rows_per_w), pl.ds(d_x, dim)],
        )

    return k(x2, pe).reshape(b, s, d_x + dim)


# speedup vs baseline: 32.3724x; 32.3724x over previous
import functools

import jax
import jax.numpy as jnp
from jax import lax
from jax.experimental import pallas as pl
from jax.experimental.pallas import tpu as pltpu
from jax.experimental.pallas import tpu_sc as plsc

_CH = 16
_NB = 4


def kernel(x, pe):
    b, s, d_x = x.shape
    size, dim = pe.shape
    d_o = d_x + dim
    rows = b * s
    x2 = x.reshape(rows, d_x)
    info = plsc.get_sparse_core_info()
    nw = info.num_cores * info.num_subcores
    rpw = rows // nw
    n_ch = rpw // _CH
    mesh = plsc.VectorSubcoreMesh(core_axis_name="c", subcore_axis_name="s")

    @functools.partial(
        pl.kernel,
        mesh=mesh,
        out_type=jax.ShapeDtypeStruct((rows, d_o), x.dtype),
        scratch_types=[pltpu.VMEM((_NB, _CH, d_o), x.dtype)]
        + [pltpu.SemaphoreType.DMA] * (2 * _NB),
    )
    def k(x_hbm, pe_hbm, out_hbm, obuf, *sems):
        in_sems, out_sems = sems[:_NB], sems[_NB:]
        wid = lax.axis_index("s") * info.num_cores + lax.axis_index("c")
        base = wid * rpw
        s_base = lax.rem(base, size)

        def start_in(ck, slot):
            r = base + ck * _CH
            sr = s_base + ck * _CH
            pltpu.async_copy(
                x_hbm.at[pl.ds(r, _CH), :],
                obuf.at[slot, :, pl.ds(0, d_x)],
                in_sems[slot],
            )
            pltpu.async_copy(
                pe_hbm.at[pl.ds(sr, _CH), :],
                obuf.at[slot, :, pl.ds(d_x, dim)],
                in_sems[slot],
            )

        def wait_in(slot):
            pltpu.make_async_copy(
                x_hbm.at[pl.ds(0, _CH), :],
                obuf.at[slot, :, pl.ds(0, d_x)],
                in_sems[slot],
            ).wait()
            pltpu.make_async_copy(
                pe_hbm.at[pl.ds(0, _CH), :],
                obuf.at[slot, :, pl.ds(d_x, dim)],
                in_sems[slot],
            ).wait()

        def start_out(ck, slot):
            r = base + ck * _CH
            pltpu.async_copy(
                obuf.at[slot], out_hbm.at[pl.ds(r, _CH), :], out_sems[slot]
            )

        def wait_out(slot):
            pltpu.make_async_copy(
                obuf.at[slot], out_hbm.at[pl.ds(0, _CH), :], out_sems[slot]
            ).wait()

        start_in(0, 0)
        start_in(1, 1)

        @pl.loop(0, n_ch, step=_NB)
        def _(c):
            for j in range(_NB):
                ck = c + j

                @pl.when(ck + 2 < n_ch)
                def _():
                    slot2 = (j + 2) % _NB

                    @pl.when(ck >= 2)
                    def _():
                        wait_out(slot2)

                    start_in(ck + 2, slot2)

                wait_in(j)
                start_out(ck, j)

        wait_out((n_ch - 2) % _NB)
        wait_out((n_ch - 1) % _NB)

    return k(x2, pe).reshape(b, s, d_o)


# speedup vs baseline: 51.4910x; 1.5906x over previous
import jax
import jax.numpy as jnp
from jax.experimental import pallas as pl

_D_X = 1024
_S_BLK = 2048


def _concat_body(x_ref, pe_ref, o_ref):
    j = pl.program_id(1)
    o_ref[:, :, :_D_X] = x_ref[...]
    o_ref[:, :, _D_X:] = pe_ref[pl.ds(j * _S_BLK, _S_BLK), :][None, :, :]


def kernel(x, pe):
    b, s, d_x = x.shape
    size, dim = pe.shape
    grid = (b, s // _S_BLK)
    return pl.pallas_call(
        _concat_body,
        grid=grid,
        in_specs=[
            pl.BlockSpec((1, _S_BLK, d_x), lambda i, j: (i, j, 0)),
            pl.BlockSpec((size, dim), lambda i, j: (0, 0)),
        ],
        out_specs=pl.BlockSpec((1, _S_BLK, d_x + dim), lambda i, j: (i, j, 0)),
        out_shape=jax.ShapeDtypeStruct((b, s, d_x + dim), x.dtype),
    )(x, pe)
